# speedup vs baseline: 5.5859x; 5.5859x over previous
import jax
import jax.numpy as jnp
from jax import lax
from jax.experimental import pallas as pl
from jax.experimental.pallas import tpu as pltpu
from jax.experimental.pallas import tpu_sc as plsc

VOCAB = 100000
EMB = 128
BATCH = 4096
SEQ = 50
HIDDEN = 64
OUT = 10

NUM_CORES = 2
NUM_SUBCORES = 16
NW = NUM_CORES * NUM_SUBCORES
ROWS_PER_W = BATCH // NW
G = 2
IDX_PER_G = G * SEQ
NGATHER = ROWS_PER_W // G
LANES = 16
NCH = EMB // LANES


def _sc_body(x_hbm, table_hbm, out_hbm, idx_v, buf0, buf1, outbuf, sem0, sem1):
  wid = lax.axis_index("s") * NUM_CORES + lax.axis_index("c")
  base = wid * ROWS_PER_W

  pltpu.sync_copy(x_hbm.at[wid], idx_v)

  pltpu.async_copy(table_hbm.at[idx_v.at[0]], buf0, sem0)
  pltpu.async_copy(table_hbm.at[idx_v.at[1]], buf1, sem1)

  def consume(buf, g):
    for r in range(G):
      accs = [jnp.zeros((LANES,), jnp.float32) for _ in range(NCH)]
      for j in range(SEQ):
        row = r * SEQ + j
        for c in range(NCH):
          accs[c] = accs[c] + buf[row, pl.ds(c * LANES, LANES)]
      for c in range(NCH):
        outbuf[g * G + r, pl.ds(c * LANES, LANES)] = accs[c]

  def loop_body(k, _):
    pltpu.make_async_copy(table_hbm.at[idx_v.at[0]], buf0, sem0).wait()
    consume(buf0, 2 * k)

    @pl.when(k < NGATHER // 2 - 1)
    def _():
      pltpu.async_copy(table_hbm.at[idx_v.at[2 * k + 2]], buf0, sem0)

    pltpu.make_async_copy(table_hbm.at[idx_v.at[1]], buf1, sem1).wait()
    consume(buf1, 2 * k + 1)

    @pl.when(k < NGATHER // 2 - 1)
    def _():
      pltpu.async_copy(table_hbm.at[idx_v.at[2 * k + 3]], buf1, sem1)

    return 0

  lax.fori_loop(0, NGATHER // 2, loop_body, 0)

  pltpu.sync_copy(outbuf, out_hbm.at[pl.ds(base, ROWS_PER_W), :])


def _sc_gather_sum(x3, table):
  mesh = plsc.VectorSubcoreMesh(core_axis_name="c", subcore_axis_name="s")
  k = pl.kernel(
      _sc_body,
      out_type=jax.ShapeDtypeStruct((BATCH, EMB), jnp.float32),
      mesh=mesh,
      scratch_types=[
          pltpu.VMEM((NGATHER, IDX_PER_G), jnp.int32),
          pltpu.VMEM((IDX_PER_G, EMB), jnp.float32),
          pltpu.VMEM((IDX_PER_G, EMB), jnp.float32),
          pltpu.VMEM((ROWS_PER_W, EMB), jnp.float32),
          pltpu.SemaphoreType.DMA,
          pltpu.SemaphoreType.DMA,
      ],
  )
  return k(x3, table)


def _mlp_body(sums_ref, len_ref, w1_ref, b1_ref, w2_ref, b2_ref, out_ref):
  s = sums_ref[...]
  inv = 1.0 / len_ref[...].astype(jnp.float32)
  rep = s * inv
  h = lax.dot_general(rep, w1_ref[...], (((1,), (1,)), ((), ())),
                      preferred_element_type=jnp.float32)
  h = jnp.maximum(h + b1_ref[...], 0.0)
  o = lax.dot_general(h, w2_ref[...], (((1,), (1,)), ((), ())),
                      preferred_element_type=jnp.float32)
  out_ref[...] = o + b2_ref[...]


def _tc_mlp(sums, lengths2, W1, b1, W2, b2):
  return pl.pallas_call(
      _mlp_body,
      out_shape=jax.ShapeDtypeStruct((BATCH, OUT), jnp.float32),
  )(sums, lengths2, W1, b1.reshape(1, HIDDEN), W2, b2.reshape(1, OUT))


def kernel(x, lengths, table, W1, b1, W2, b2):
  x3 = x.reshape(NW, NGATHER, IDX_PER_G)
  sums = _sc_gather_sum(x3, table)
  return _tc_mlp(sums, lengths.reshape(BATCH, 1), W1, b1, W2, b2)


# speedup vs baseline: 14.2715x; 2.5549x over previous
import jax
import jax.numpy as jnp
from jax import lax
from jax.experimental import pallas as pl
from jax.experimental.pallas import tpu as pltpu
from jax.experimental.pallas import tpu_sc as plsc

VOCAB = 100000
EMB = 128
BATCH = 4096
SEQ = 50
HIDDEN = 64
OUT = 10

NUM_CORES = 2
NUM_SUBCORES = 16
NW = NUM_CORES * NUM_SUBCORES
ROWS_PER_W = BATCH // NW
LANES = 16
NCH = EMB // LANES


def _sc_body(xt_hbm, table_hbm, out_hbm, idx_v, acc, sem):
  wid = lax.axis_index("s") * NUM_CORES + lax.axis_index("c")
  base = wid * ROWS_PER_W

  pltpu.sync_copy(xt_hbm.at[wid], idx_v)

  zeros = jnp.zeros((LANES,), jnp.float32)

  def zero_body(r, _):
    for c in range(NCH):
      acc[r, pl.ds(c * LANES, LANES)] = zeros
    return 0

  lax.fori_loop(0, ROWS_PER_W, zero_body, 0)

  def fire(j, _):
    pltpu.async_copy(table_hbm.at[idx_v.at[j]], acc, sem, add=True)
    return 0

  lax.fori_loop(0, SEQ, fire, 0)

  def drain(j, _):
    pltpu.make_async_copy(table_hbm.at[idx_v.at[0]], acc, sem).wait()
    return 0

  lax.fori_loop(0, SEQ, drain, 0)

  pltpu.sync_copy(acc, out_hbm.at[pl.ds(base, ROWS_PER_W), :])


def _sc_gather_sum(xt, table):
  mesh = plsc.VectorSubcoreMesh(core_axis_name="c", subcore_axis_name="s")
  k = pl.kernel(
      _sc_body,
      out_type=jax.ShapeDtypeStruct((BATCH, EMB), jnp.float32),
      mesh=mesh,
      scratch_types=[
          pltpu.VMEM((SEQ, ROWS_PER_W), jnp.int32),
          pltpu.VMEM((ROWS_PER_W, EMB), jnp.float32),
          pltpu.SemaphoreType.DMA,
      ],
  )
  return k(xt, table)


def _mlp_body(sums_ref, len_ref, w1_ref, b1_ref, w2_ref, b2_ref, out_ref):
  s = sums_ref[...]
  inv = 1.0 / len_ref[...].astype(jnp.float32)
  rep = s * inv
  h = lax.dot_general(rep, w1_ref[...], (((1,), (1,)), ((), ())),
                      preferred_element_type=jnp.float32)
  h = jnp.maximum(h + b1_ref[...], 0.0)
  o = lax.dot_general(h, w2_ref[...], (((1,), (1,)), ((), ())),
                      preferred_element_type=jnp.float32)
  out_ref[...] = o + b2_ref[...]


def _tc_mlp(sums, lengths2, W1, b1, W2, b2):
  return pl.pallas_call(
      _mlp_body,
      out_shape=jax.ShapeDtypeStruct((BATCH, OUT), jnp.float32),
  )(sums, lengths2, W1, b1.reshape(1, HIDDEN), W2, b2.reshape(1, OUT))


def kernel(x, lengths, table, W1, b1, W2, b2):
  xt = x.reshape(NW, ROWS_PER_W, SEQ).transpose(0, 2, 1)
  sums = _sc_gather_sum(xt, table)
  return _tc_mlp(sums, lengths.reshape(BATCH, 1), W1, b1, W2, b2)
